# baseline (speedup 1.0000x reference)
import jax
import jax.numpy as jnp
from jax import lax
from jax.experimental import pallas as pl
from jax.experimental.pallas import tpu as pltpu

_EPS2 = 1e-24


def _silu(y):
    return y * jax.nn.sigmoid(y)


def _conv_norm_silu(y_ref, wp_ref, wh_ref, b_ref, g_ref, ho, wo, ci, co,
                    pad_edges):
    m = ho * wo
    c2 = 2 * ci
    acc = jnp.zeros((m, co), jnp.float32)
    for kh in range(4):
        base, par = kh // 2, kh % 2
        xs = y_ref[base:base + ho, par, 1:1 + wo, :].reshape(m, c2)
        acc = acc + jnp.dot(xs, wp_ref[kh],
                            preferred_element_type=jnp.float32)
        if pad_edges:
            x0 = y_ref[base:base + ho, par, 0:wo, :].reshape(m, c2)
            x2 = y_ref[base:base + ho, par, 2:2 + wo, :].reshape(m, c2)
        else:
            x0 = y_ref[base:base + ho, par, 0:wo, ci:].reshape(m, ci)
            x2 = y_ref[base:base + ho, par, 2:2 + wo, :ci].reshape(m, ci)
        acc = acc + jnp.dot(x0, wh_ref[2 * kh],
                            preferred_element_type=jnp.float32)
        acc = acc + jnp.dot(x2, wh_ref[2 * kh + 1],
                            preferred_element_type=jnp.float32)
    z = acc + b_ref[...]
    ss = jnp.sum(z * z, axis=1, keepdims=True)
    inv = lax.rsqrt(jnp.maximum(ss, _EPS2))
    return _silu(z * inv * g_ref[...])


def _make_block_body(ho, wo, ci, co, pad_edges):
    wo2 = wo // 2

    def body(y_ref, wp_ref, wh_ref, b_ref, g_ref, o_ref):
        r = _conv_norm_silu(y_ref, wp_ref, wh_ref, b_ref, g_ref,
                            ho, wo, ci, co, pad_edges)
        o_ref[...] = jnp.zeros(o_ref.shape, o_ref.dtype)
        o_ref[1:ho + 1, 1:wo2 + 1, :] = (
            r.reshape(ho, wo2, 2 * co).astype(o_ref.dtype))
    return body


def _make_block2_body(ho, wo, ci, co, pad_edges):
    def body(y_ref, wp_ref, wh_ref, b_ref, g_ref, w1_ref, b1_ref, o_ref):
        r = _conv_norm_silu(y_ref, wp_ref, wh_ref, b_ref, g_ref,
                            ho, wo, ci, co, pad_edges)
        z = jnp.dot(r.astype(w1_ref.dtype), w1_ref[...],
                    preferred_element_type=jnp.float32) + b1_ref[...]
        o_ref[...] = _silu(z).astype(o_ref.dtype)
    return body


def _block(y, w, b, g, fuse1x1=None):
    n, hcells, _, wc, c2 = y.shape
    ho, wo, ci = hcells - 1, wc - 2, c2 // 2
    co = w.shape[0]
    m = ho * wo
    pad_edges = ci < 128

    wT = w.transpose(2, 3, 1, 0).astype(jnp.bfloat16)
    wp = jnp.stack([jnp.concatenate([wT[kh, 1], wT[kh, 2]], axis=0)
                    for kh in range(4)])
    zpad = jnp.zeros((ci, co), jnp.bfloat16)
    wh = []
    for kh in range(4):
        if pad_edges:
            wh.append(jnp.concatenate([zpad, wT[kh, 0]], axis=0))
            wh.append(jnp.concatenate([wT[kh, 3], zpad], axis=0))
        else:
            wh.append(wT[kh, 0])
            wh.append(wT[kh, 3])
    wh = jnp.stack(wh)
    kdim = wh.shape[1]
    bb = b.astype(jnp.float32).reshape(1, co)
    gg = ((float(co) ** 0.5) * (g.astype(jnp.float32) + 1.0)).reshape(1, co)

    in_specs = [
        pl.BlockSpec((None, hcells, 2, wc, c2), lambda i: (i, 0, 0, 0, 0)),
        pl.BlockSpec((4, 2 * ci, co), lambda i: (0, 0, 0)),
        pl.BlockSpec((8, kdim, co), lambda i: (0, 0, 0)),
        pl.BlockSpec((1, co), lambda i: (0, 0)),
        pl.BlockSpec((1, co), lambda i: (0, 0)),
    ]
    args = [y, wp, wh, bb, gg]
    flops = 2 * n * m * (16 + (8 if pad_edges else 0)) * ci * co
    if fuse1x1 is None:
        body = _make_block_body(ho, wo, ci, co, pad_edges)
        out_shape = jax.ShapeDtypeStruct((n, ho + 2, wo // 2 + 2, 2 * co),
                                         jnp.bfloat16)
        out_spec = pl.BlockSpec((None, ho + 2, wo // 2 + 2, 2 * co),
                                lambda i: (i, 0, 0, 0))
    else:
        w1, b1 = fuse1x1
        w1m = w1.reshape(co, co).T.astype(jnp.bfloat16)
        in_specs += [
            pl.BlockSpec((co, co), lambda i: (0, 0)),
            pl.BlockSpec((1, co), lambda i: (0, 0)),
        ]
        args += [w1m, b1.astype(jnp.float32).reshape(1, co)]
        flops += 2 * n * m * co * co
        body = _make_block2_body(ho, wo, ci, co, pad_edges)
        out_shape = jax.ShapeDtypeStruct((n, m, co), jnp.bfloat16)
        out_spec = pl.BlockSpec((None, m, co), lambda i: (i, 0, 0))

    return pl.pallas_call(
        body,
        out_shape=out_shape,
        grid=(n,),
        in_specs=in_specs,
        out_specs=out_spec,
        compiler_params=pltpu.CompilerParams(
            dimension_semantics=("parallel",),
        ),
        cost_estimate=pl.CostEstimate(
            flops=flops,
            transcendentals=2 * n * m * co,
            bytes_accessed=(y.size + n * m * co) * 2,
        ),
    )(*args)


def kernel(layer0_w, layer0_b, block0_w, block0_b, block0_g,
           block1_w, block1_b, block1_g, block2_w, block2_b, block2_g,
           logits_w1, logits_b1, logits_w2, logits_b2, x):
    n, _, hh, ww = x.shape
    y0 = lax.conv_general_dilated(
        x.astype(jnp.bfloat16), layer0_w.astype(jnp.bfloat16),
        window_strides=(1, 1), padding=((2, 2), (2, 2)),
        dimension_numbers=("NCHW", "OIHW", "NHWC"),
        preferred_element_type=jnp.float32)
    c = y0.shape[-1]
    y0 = _silu(y0 + layer0_b).astype(jnp.bfloat16)
    y0 = jnp.pad(y0, ((0, 0), (1, 1), (2, 2), (0, 0)))
    y0 = y0.reshape(n, hh // 2 + 1, 2, ww // 2 + 2, 2 * c)

    h = _block(y0, block0_w, block0_b, block0_g)
    h = h.reshape(n, h.shape[1] // 2, 2, h.shape[2], h.shape[3])
    h = _block(h, block1_w, block1_b, block1_g)
    h = h.reshape(n, h.shape[1] // 2, 2, h.shape[2], h.shape[3])
    h = _block(h, block2_w, block2_b, block2_g,
               fuse1x1=(logits_w1, logits_b1))
    ho, wo = hh // 8, ww // 8
    h = h.reshape(n, ho, wo, h.shape[-1])

    preds = lax.conv_general_dilated(
        h, logits_w2.astype(jnp.bfloat16),
        window_strides=(1, 1), padding="VALID",
        dimension_numbers=("NHWC", "OIHW", "NCHW"),
        preferred_element_type=jnp.float32)
    return preds + logits_b2.reshape(1, -1, 1, 1)
